# Initial kernel scaffold; baseline (speedup 1.0000x reference)
import jax
import jax.numpy as jnp
from jax.experimental import pallas as pl


def kernel(x, conv1_w, conv1_b_raw, conv2_w, conv2_b_raw, conv1_wmat, conv1_b, conv2_wmat, conv2_b, mlp_w0, mlp_b0, mlp_w1, mlp_b1, mlp_w2, mlp_b2, lin_w0, lin_b0, lin_w1, lin_b1, lin_w2, lin_b2):
    raise NotImplementedError("write your pallas kernel here")


# speedup vs baseline: 13.4278x; 13.4278x over previous
import functools

import jax
import jax.numpy as jnp
from jax.experimental import pallas as pl
from jax.experimental.pallas import tpu as pltpu


def _fused_kernel(x_ref, t1_ref, t2_ref, w0_ref, w1_ref, w2_ref,
                  bc1_ref, bc2_ref, b0_ref, b1_ref, b2_ref, o_ref,
                  *, bb, ph1, pw1, oh2, ph2, pw2, c2):
    f32 = jnp.float32
    x = x_ref[...]
    oh1 = 2 * ph1
    n1 = pw1 * 2 * 10

    y = jnp.zeros((bb * oh1, n1), f32)
    for kh in range(5):
        xk = x[:, kh:kh + oh1, :].reshape(bb * oh1, x.shape[-1])
        y = y + jnp.dot(xk, t1_ref[kh], preferred_element_type=f32)
    yv = y.reshape(bb, ph1, 2, n1)
    rm = jnp.maximum(yv[:, :, 0, :], yv[:, :, 1, :])
    m = jnp.maximum(rm[:, :, 0:n1 - 10], rm[:, :, 10:n1])
    f1 = jnp.maximum(m + bc1_ref[...], 0.0)

    n2 = oh2 * c2
    y2 = jnp.zeros((bb * oh2, n2), f32)
    for kh in range(5):
        fk = f1[:, kh:kh + oh2, :].reshape(bb * oh2, f1.shape[-1])
        y2 = y2 + jnp.dot(fk, t2_ref[kh], preferred_element_type=f32)
    y2v = y2.reshape(bb, ph2, 2, n2)
    rm2 = jnp.maximum(y2v[:, :, 0, :], y2v[:, :, 1, :])
    m2 = jnp.maximum(rm2[:, :, 0:n2 - c2], rm2[:, :, c2:n2])
    h = jnp.maximum(m2 + bc2_ref[...], 0.0)

    u = jnp.dot(h[:, 0, :], w0_ref[0], preferred_element_type=f32)
    for p in range(1, ph2):
        u = u + jnp.dot(h[:, p, :], w0_ref[p], preferred_element_type=f32)
    u = jnp.maximum(u + b0_ref[...], 0.0)
    v = jnp.dot(u, w1_ref[...], preferred_element_type=f32)
    v = jnp.maximum(v + b1_ref[...], 0.0)
    o_ref[0] = (jnp.dot(v, w2_ref[...], preferred_element_type=f32)
                + b2_ref[...]).astype(o_ref.dtype)


def kernel(x, conv1_w, conv1_b_raw, conv2_w, conv2_b_raw, conv1_wmat,
           conv1_b, conv2_wmat, conv2_b, mlp_w0, mlp_b0, mlp_w1, mlp_b1,
           mlp_w2, mlp_b2, lin_w0, lin_b0, lin_w1, lin_b1, lin_w2, lin_b2):
    f32 = jnp.float32
    nt, no = x.shape[:2]
    B = nt * no
    H = x.shape[-1]
    c1 = conv1_w.shape[0]
    c2 = conv2_w.shape[0]
    oh1 = H - 4
    ph1 = oh1 // 2
    oh2 = ph1 - 4
    ph2 = oh2 // 2
    hid1 = lin_w0.shape[0]
    hid2 = lin_w1.shape[0]
    dim_out = lin_w2.shape[0]

    x3 = x.reshape(B, H, H)

    w1m = conv1_w[:, 0]
    e1 = (jnp.arange(H)[None, :, None]
          == jnp.arange(oh1)[None, None, :] + jnp.arange(5)[:, None, None]
          ).astype(f32)
    t1 = jnp.einsum('kio,chk->hioc', e1, w1m).reshape(5, H, oh1 * c1)

    d2 = (jnp.arange(ph1)[None, :, None]
          == jnp.arange(oh2)[None, None, :] + jnp.arange(5)[:, None, None]
          ).astype(f32)
    t2a = jnp.einsum('kpo,dchk->hpcod', d2, conv2_w)
    t2 = jnp.pad(t2a, ((0, 0), (0, 0), (0, c2 - c1), (0, 0), (0, 0)))
    t2 = t2.reshape(5, ph1 * c2, oh2 * c2)[:, :ph1 * c2 - c1, :]

    w0r = lin_w0.reshape(hid1, c2, ph2, pw2 := ph2)
    w0t = jnp.transpose(w0r, (2, 3, 1, 0))
    w0p = jnp.pad(w0t, ((0, 0), (0, 0), (0, c2), (0, 0)))
    w0s = w0p.reshape(ph2, pw2 * 2 * c2, hid1)[:, :pw2 * 2 * c2 - c2, :]

    bc1 = jnp.tile(conv1_b_raw, 2 * ph1 - 1)[None]
    bc2 = jnp.tile(conv2_b_raw, 2 * ph2 - 1)[None]
    w2t = lin_w2.T
    b2t = lin_b2[None]

    nsplit = 2 if B % 2 == 0 else 1
    bb = B // nsplit
    zero = lambda i: (0, 0, 0)
    out = pl.pallas_call(
        functools.partial(_fused_kernel, bb=bb, ph1=ph1, pw1=ph1, oh2=oh2,
                          ph2=ph2, pw2=ph2, c2=c2),
        out_shape=jax.ShapeDtypeStruct((nsplit, bb, dim_out), f32),
        grid=(nsplit,),
        in_specs=[
            pl.BlockSpec((bb, H, H), lambda i: (i, 0, 0)),
            pl.BlockSpec(t1.shape, zero),
            pl.BlockSpec(t2.shape, zero),
            pl.BlockSpec(w0s.shape, zero),
            pl.BlockSpec(mlp_w1.shape, lambda i: (0, 0)),
            pl.BlockSpec(w2t.shape, lambda i: (0, 0)),
            pl.BlockSpec(bc1.shape, lambda i: (0, 0)),
            pl.BlockSpec(bc2.shape, lambda i: (0, 0)),
            pl.BlockSpec(mlp_b0.shape, lambda i: (0, 0)),
            pl.BlockSpec(mlp_b1.shape, lambda i: (0, 0)),
            pl.BlockSpec(b2t.shape, lambda i: (0, 0)),
        ],
        out_specs=pl.BlockSpec((1, bb, dim_out), lambda i: (i, 0, 0)),
        compiler_params=pltpu.CompilerParams(
            dimension_semantics=("parallel",),
            vmem_limit_bytes=50 * 1024 * 1024),
    )(x3, t1, t2, w0s, mlp_w1, w2t, bc1, bc2, mlp_b0, mlp_b1, b2t)
    return out.reshape(nt, no, dim_out)
